# Initial kernel scaffold; baseline (speedup 1.0000x reference)
import jax
import jax.numpy as jnp
from jax.experimental import pallas as pl


def kernel(co_visible, pre_corr):
    raise NotImplementedError("write your pallas kernel here")


# speedup vs baseline: 38.4592x; 38.4592x over previous
import jax
import jax.numpy as jnp
from jax import lax
from jax.experimental import pallas as pl

_THR = 0.3
_TOPK = 16
_H = 128
_W = 128
_N = _TOPK * _H * _W


def _batch_kernel(co_ref, pre_ref, out_ref):
    cin = co_ref[0, 0]
    cref = co_ref[0, 1]

    in_mask = cin > _THR
    incnt = jnp.sum(in_mask.astype(jnp.int32))
    refcnt = jnp.sum((cref > _THR).astype(jnp.int32))
    k_sel = jnp.minimum(jnp.maximum(refcnt, 1), _TOPK * incnt)

    vals = pre_ref[0]
    u = lax.bitcast_convert_type(vals + 0.0, jnp.uint32)
    neg = (u >> jnp.uint32(31)) > jnp.uint32(0)
    key = jnp.where(neg, ~u, u | jnp.uint32(0x80000000))

    mask3 = jnp.broadcast_to(in_mask[None, :, :], (_TOPK, _H, _W))

    def count_ge(cand):
        m = mask3 & (key >= cand)
        return jnp.sum(m.astype(jnp.int32))

    def bit_body(i, prefix):
        cand = prefix | (jnp.uint32(1) << (jnp.uint32(31) - i.astype(jnp.uint32)))
        c = count_ge(cand)
        return jnp.where(c >= k_sel, cand, prefix)

    t_key = lax.fori_loop(0, 32, bit_body, jnp.uint32(0))

    greater = jnp.sum((mask3 & (key > t_key)).astype(jnp.int32))
    budget = k_sel - greater

    ti = lax.broadcasted_iota(jnp.int32, (_TOPK, _H, _W), 0)
    hi = lax.broadcasted_iota(jnp.int32, (_TOPK, _H, _W), 1)
    wi = lax.broadcasted_iota(jnp.int32, (_TOPK, _H, _W), 2)
    idx3 = ti * (_H * _W) + hi * _W + wi

    ties = mask3 & (key == t_key)

    def idx_body(i, pref):
        cand = pref | (jnp.int32(1) << (jnp.int32(17) - i))
        c = jnp.sum((ties & (idx3 < cand)).astype(jnp.int32))
        return jnp.where(c < budget, cand, pref)

    i_cut = lax.fori_loop(0, 18, idx_body, jnp.int32(0))

    sel = mask3 & ((key > t_key) | (ties & (idx3 <= i_cut)))
    sel = sel | (ti == 0)
    sel_b = sel.astype(jnp.bfloat16)

    w2 = lax.broadcasted_iota(jnp.int32, (_W, 4 * _W), 0)
    c2 = lax.broadcasted_iota(jnp.int32, (_W, 4 * _W), 1)
    e_cols = ((c2 // 4) == w2).astype(jnp.bfloat16)
    r2 = lax.broadcasted_iota(jnp.int32, (4 * _H, _H), 0)
    h2 = lax.broadcasted_iota(jnp.int32, (4 * _H, _H), 1)
    f_rows = ((r2 // 4) == h2).astype(jnp.bfloat16)

    for t in range(_TOPK):
        a = jnp.dot(f_rows, sel_b[t], preferred_element_type=jnp.float32)
        b = jnp.dot(a.astype(jnp.bfloat16), e_cols,
                    preferred_element_type=jnp.float32)
        out_ref[0, t] = b > 0.5


def kernel(co_visible, pre_corr):
    batch = pre_corr.shape[0]
    return pl.pallas_call(
        _batch_kernel,
        grid=(batch,),
        in_specs=[
            pl.BlockSpec((1, 2, _H, _W), lambda b: (b, 0, 0, 0)),
            pl.BlockSpec((1, _TOPK, _H, _W), lambda b: (b, 0, 0, 0)),
        ],
        out_specs=pl.BlockSpec((1, _TOPK, 4 * _H, 4 * _W),
                               lambda b: (b, 0, 0, 0)),
        out_shape=jax.ShapeDtypeStruct((batch, _TOPK, 4 * _H, 4 * _W),
                                       jnp.bool_),
    )(co_visible, pre_corr)


# speedup vs baseline: 57.4943x; 1.4949x over previous
import jax
import jax.numpy as jnp
from jax import lax
from jax.experimental import pallas as pl
from jax.experimental.pallas import tpu as pltpu

_THR = 0.3
_TOPK = 16
_H = 128
_W = 128
_B = 4


def _kernel_body(co_ref, pre_ref, out_ref, sel_ref):
    b = pl.program_id(0)

    @pl.when(b == 0)
    def _phase1():
        cin = co_ref[:, 0]
        cref = co_ref[:, 1]

        def count2(x):
            return jnp.sum(jnp.sum(x, axis=1), axis=1)

        in_mask = cin > _THR
        incnt = count2(in_mask.astype(jnp.int32))
        refcnt = count2((cref > _THR).astype(jnp.int32))
        k_sel = jnp.minimum(jnp.maximum(refcnt, 1), _TOPK * incnt)

        vals = pre_ref[...]
        u = lax.bitcast_convert_type(vals + 0.0, jnp.uint32)
        neg = (u >> jnp.uint32(31)) > jnp.uint32(0)
        key = jnp.where(neg, ~u, u | jnp.uint32(0x80000000))
        mask4 = jnp.broadcast_to(in_mask[:, None, :, :], (_B, _TOPK, _H, _W))
        mkey = jnp.where(mask4, key, jnp.uint32(0))

        def count4(m):
            s = jnp.sum(m.astype(jnp.int32), axis=(1, 2))
            return jnp.sum(s, axis=1)

        def bit_body(i, prefix):
            cand = prefix | (jnp.uint32(1) << (jnp.uint32(31) - i.astype(jnp.uint32)))
            c = count4(mkey >= cand[:, None, None, None])
            return jnp.where(c >= k_sel, cand, prefix)

        t_key = lax.fori_loop(0, 32, bit_body, jnp.zeros((_B,), jnp.uint32))

        greater = count4(mkey > t_key[:, None, None, None])
        budget = k_sel - greater

        ti = lax.broadcasted_iota(jnp.int32, (_B, _TOPK, _H, _W), 1)
        hi = lax.broadcasted_iota(jnp.int32, (_B, _TOPK, _H, _W), 2)
        wi = lax.broadcasted_iota(jnp.int32, (_B, _TOPK, _H, _W), 3)
        idx4 = ti * (_H * _W) + hi * _W + wi

        ties = mkey == t_key[:, None, None, None]

        def idx_body(i, pref):
            cand = pref | (jnp.int32(1) << (jnp.int32(17) - i))
            c = count4(ties & (idx4 < cand[:, None, None, None]))
            return jnp.where(c < budget, cand, pref)

        i_cut = lax.fori_loop(0, 18, idx_body, jnp.zeros((_B,), jnp.int32))

        sel = (mkey > t_key[:, None, None, None]) | (
            ties & (idx4 <= i_cut[:, None, None, None]))
        sel = sel | (ti == 0)
        sel_ref[...] = sel.astype(jnp.int8)

    w2 = lax.broadcasted_iota(jnp.int32, (_W, 4 * _W), 0)
    c2 = lax.broadcasted_iota(jnp.int32, (_W, 4 * _W), 1)
    e_cols = ((c2 // 4) == w2).astype(jnp.bfloat16)
    r2 = lax.broadcasted_iota(jnp.int32, (4 * _H, _H), 0)
    h2 = lax.broadcasted_iota(jnp.int32, (4 * _H, _H), 1)
    f_rows = ((r2 // 4) == h2).astype(jnp.bfloat16)

    sel_b = sel_ref[b].astype(jnp.bfloat16)
    for t in range(_TOPK):
        a = jnp.dot(f_rows, sel_b[t], preferred_element_type=jnp.float32)
        o = jnp.dot(a.astype(jnp.bfloat16), e_cols,
                    preferred_element_type=jnp.float32)
        out_ref[0, t] = o > 0.5


def kernel(co_visible, pre_corr):
    return pl.pallas_call(
        _kernel_body,
        grid=(_B,),
        in_specs=[
            pl.BlockSpec((_B, 2, _H, _W), lambda b: (0, 0, 0, 0)),
            pl.BlockSpec((_B, _TOPK, _H, _W), lambda b: (0, 0, 0, 0)),
        ],
        out_specs=pl.BlockSpec((1, _TOPK, 4 * _H, 4 * _W),
                               lambda b: (b, 0, 0, 0)),
        out_shape=jax.ShapeDtypeStruct((_B, _TOPK, 4 * _H, 4 * _W),
                                       jnp.bool_),
        scratch_shapes=[pltpu.VMEM((_B, _TOPK, _H, _W), jnp.int8)],
    )(co_visible, pre_corr)


# speedup vs baseline: 67.5204x; 1.1744x over previous
import jax
import jax.numpy as jnp
from jax import lax
from jax.experimental import pallas as pl
from jax.experimental.pallas import tpu as pltpu

_THR = 0.3
_TOPK = 16
_H = 128
_W = 128
_B = 4


def _kernel_body(co_ref, pre_ref, out_ref, sel_ref):
    b = pl.program_id(0)

    @pl.when(b == 0)
    def _phase1():
        cin = co_ref[:, 0]
        cref = co_ref[:, 1]

        def count2(x):
            return jnp.sum(jnp.sum(x, axis=1), axis=1)

        in_mask = cin > _THR
        incnt = count2(in_mask.astype(jnp.int32))
        refcnt = count2((cref > _THR).astype(jnp.int32))
        k_sel = jnp.minimum(jnp.maximum(refcnt, 1), _TOPK * incnt)

        vals = pre_ref[...]
        u = lax.bitcast_convert_type(vals + 0.0, jnp.uint32)
        neg = (u >> jnp.uint32(31)) > jnp.uint32(0)
        key = jnp.where(neg, ~u, u | jnp.uint32(0x80000000))
        mask4 = jnp.broadcast_to(in_mask[:, None, :, :], (_B, _TOPK, _H, _W))
        mkey = jnp.where(mask4, key, jnp.uint32(0))

        rows = _B * _TOPK * _H
        bi = lax.broadcasted_iota(jnp.int32, (_B, rows), 0)
        ri = lax.broadcasted_iota(jnp.int32, (_B, rows), 1)
        ones_blk = ((ri // (_TOPK * _H)) == bi).astype(jnp.bfloat16)

        def count4(m):
            mb = m.astype(jnp.bfloat16).reshape(rows, _W)
            s = jnp.dot(ones_blk, mb, preferred_element_type=jnp.float32)
            return jnp.sum(s, axis=1)

        k_f = k_sel.astype(jnp.float32)

        def bit_body(i, prefix):
            cand = prefix | (jnp.uint32(1) << (jnp.uint32(31) - i.astype(jnp.uint32)))
            c = count4(mkey >= cand[:, None, None, None])
            return jnp.where(c >= k_f, cand, prefix)

        t_key = lax.fori_loop(0, 32, bit_body, jnp.zeros((_B,), jnp.uint32))

        greater = count4(mkey > t_key[:, None, None, None])
        budget = k_f - greater

        ti = lax.broadcasted_iota(jnp.int32, (_B, _TOPK, _H, _W), 1)
        hi = lax.broadcasted_iota(jnp.int32, (_B, _TOPK, _H, _W), 2)
        wi = lax.broadcasted_iota(jnp.int32, (_B, _TOPK, _H, _W), 3)
        idx4 = ti * (_H * _W) + hi * _W + wi

        ties = mkey == t_key[:, None, None, None]
        tiecnt = count4(ties)

        def idx_loop(_):
            def idx_body(i, pref):
                cand = pref | (jnp.int32(1) << (jnp.int32(17) - i))
                c = count4(ties & (idx4 < cand[:, None, None, None]))
                return jnp.where(c < budget, cand, pref)

            return lax.fori_loop(0, 18, idx_body, jnp.zeros((_B,), jnp.int32))

        i_cut = lax.cond(
            jnp.any(tiecnt > budget), idx_loop,
            lambda _: jnp.full((_B,), _TOPK * _H * _W - 1, jnp.int32),
            operand=None)

        sel = (mkey > t_key[:, None, None, None]) | (
            ties & (idx4 <= i_cut[:, None, None, None]))
        sel = sel | (ti == 0)
        sel_ref[...] = sel.astype(jnp.int8)

    w2 = lax.broadcasted_iota(jnp.int32, (_W, 4 * _W), 0)
    c2 = lax.broadcasted_iota(jnp.int32, (_W, 4 * _W), 1)
    e_cols = ((c2 // 4) == w2).astype(jnp.bfloat16)
    r2 = lax.broadcasted_iota(jnp.int32, (4 * _H, _H), 0)
    h2 = lax.broadcasted_iota(jnp.int32, (4 * _H, _H), 1)
    f_rows = ((r2 // 4) == h2).astype(jnp.bfloat16)

    sel_b = sel_ref[b].astype(jnp.bfloat16)
    for t in range(_TOPK):
        a = jnp.dot(f_rows, sel_b[t], preferred_element_type=jnp.float32)
        o = jnp.dot(a.astype(jnp.bfloat16), e_cols,
                    preferred_element_type=jnp.float32)
        out_ref[0, t] = o > 0.5


def kernel(co_visible, pre_corr):
    return pl.pallas_call(
        _kernel_body,
        grid=(_B,),
        in_specs=[
            pl.BlockSpec((_B, 2, _H, _W), lambda b: (0, 0, 0, 0)),
            pl.BlockSpec((_B, _TOPK, _H, _W), lambda b: (0, 0, 0, 0)),
        ],
        out_specs=pl.BlockSpec((1, _TOPK, 4 * _H, 4 * _W),
                               lambda b: (b, 0, 0, 0)),
        out_shape=jax.ShapeDtypeStruct((_B, _TOPK, 4 * _H, 4 * _W),
                                       jnp.bool_),
        scratch_shapes=[pltpu.VMEM((_B, _TOPK, _H, _W), jnp.int8)],
    )(co_visible, pre_corr)
